# Initial kernel scaffold; baseline (speedup 1.0000x reference)
import jax
import jax.numpy as jnp
from jax.experimental import pallas as pl


def kernel(center_table, context_table, center_ids, context_ids, neg_context_ids):
    raise NotImplementedError("write your pallas kernel here")


# speedup vs baseline: 1.6813x; 1.6813x over previous
import functools

import jax
import jax.numpy as jnp
from jax import lax
from jax.experimental import pallas as pl
from jax.experimental.pallas import tpu as pltpu
from jax.experimental.pallas import tpu_sc as plsc

D = 64
B = 16384
K = 5
NC = 2
NS = 16
NW = NC * NS
PER_W = B // NW
C = 128
NCHUNK = PER_W // C


def _softplus(u):
    t = jnp.exp(-jnp.abs(u))
    s = t / (t + 2.0)
    p = s * s
    ser = s * (1.0 + p * (1.0 / 3.0 + p * (1.0 / 5.0 + p * (1.0 / 7.0 + p * (1.0 / 9.0)))))
    return jnp.maximum(u, 0.0) + 2.0 * ser


def _body(center_hbm, context_hbm, cids_hbm, xids_hbm, negt_hbm, out_hbm,
          cidx, xidx, nidx, crow, xrow, nrow, stage, accv, sem):
    wid = lax.axis_index("s") * NC + lax.axis_index("c")

    def chunk(j, acc):
        base = wid * PER_W + j * C
        pltpu.sync_copy(cids_hbm.at[pl.ds(base, C)], cidx)
        pltpu.sync_copy(xids_hbm.at[pl.ds(base, C)], xidx)
        for k in range(K):
            pltpu.sync_copy(negt_hbm.at[pl.ds(k * B + base, C)], nidx.at[k])
        cps = [pltpu.async_copy(center_hbm.at[cidx], crow, sem),
               pltpu.async_copy(context_hbm.at[xidx], xrow, sem)]
        for k in range(K):
            cps.append(pltpu.async_copy(context_hbm.at[nidx.at[k]], nrow.at[k], sem))
        for cp in cps:
            cp.wait()

        def elem(e, carry):
            c0 = crow[e, pl.ds(0, 16)]
            c1 = crow[e, pl.ds(16, 16)]
            c2 = crow[e, pl.ds(32, 16)]
            c3 = crow[e, pl.ds(48, 16)]
            x0 = xrow[e, pl.ds(0, 16)]
            x1 = xrow[e, pl.ds(16, 16)]
            x2 = xrow[e, pl.ds(32, 16)]
            x3 = xrow[e, pl.ds(48, 16)]
            pos = c0 * x0 + c1 * x1 + c2 * x2 + c3 * x3
            base_s = e * (1 + K) * 16
            stage[pl.ds(base_s, 16)] = plsc.cumsum(-pos)
            for k in range(K):
                n0 = nrow[k, e, pl.ds(0, 16)]
                n1 = nrow[k, e, pl.ds(16, 16)]
                n2 = nrow[k, e, pl.ds(32, 16)]
                n3 = nrow[k, e, pl.ds(48, 16)]
                neg = n0 * c0 + n1 * c1 + n2 * c2 + n3 * c3
                stage[pl.ds(base_s + (1 + k) * 16, 16)] = plsc.cumsum(neg)
            return carry

        acc = lax.fori_loop(0, C, elem, acc)

        lane = lax.iota(jnp.int32, 16)

        def epi(g, a):
            r = g % (1 + K)
            grp = g // (1 + K)
            idx = lane * ((1 + K) * 16) + (grp * 16 * (1 + K) * 16 + r * 16 + 15)
            v = plsc.load_gather(stage, [idx])
            u = jnp.clip(v, -10.0, 10.0)
            return a + _softplus(u)

        return lax.fori_loop(0, (1 + K) * C // 16, epi, acc)

    acc = lax.fori_loop(0, NCHUNK, chunk, jnp.zeros((16,), jnp.float32))
    accv[...] = acc
    pltpu.sync_copy(accv, out_hbm.at[wid])


@jax.jit
def _sc_loss_partials(center_table, context_table, center_ids, context_ids, negt):
    mesh = plsc.VectorSubcoreMesh(core_axis_name="c", subcore_axis_name="s")
    f = pl.kernel(
        _body,
        out_type=jax.ShapeDtypeStruct((NW, 16), jnp.float32),
        mesh=mesh,
        compiler_params=pltpu.CompilerParams(
            needs_layout_passes=False, use_tc_tiling_on_sc=False),
        scratch_types=[
            pltpu.VMEM((C,), jnp.int32),
            pltpu.VMEM((C,), jnp.int32),
            pltpu.VMEM((K, C), jnp.int32),
            pltpu.VMEM((C, D), jnp.float32),
            pltpu.VMEM((C, D), jnp.float32),
            pltpu.VMEM((K, C, D), jnp.float32),
            pltpu.VMEM(((1 + K) * C * 16,), jnp.float32),
            pltpu.VMEM((16,), jnp.float32),
            pltpu.SemaphoreType.DMA,
        ],
    )
    return f(center_table, context_table, center_ids, context_ids, negt)


def kernel(center_table, context_table, center_ids, context_ids, neg_context_ids):
    negt = neg_context_ids.T.reshape(-1)
    partials = _sc_loss_partials(center_table, context_table, center_ids,
                                 context_ids, negt)
    return jnp.sum(partials) / B


# speedup vs baseline: 1.7569x; 1.0450x over previous
import jax
import jax.numpy as jnp
from jax import lax
from jax.experimental import pallas as pl
from jax.experimental.pallas import tpu as pltpu
from jax.experimental.pallas import tpu_sc as plsc

D = 64
B = 16384
K = 5
NC = 2
NS = 16
NW = NC * NS
PER_W = B // NW
C = 64
NCHUNK = PER_W // C
R = 1 + K
NEG_SPLITS = [(0, 128), (128, 128), (256, 64)]


def _softplus(u):
    t = jnp.exp(-jnp.abs(u))
    s = t / (t + 2.0)
    p = s * s
    ser = s * (1.0 + p * (1.0 / 3.0 + p * (1.0 / 5.0 + p * (1.0 / 7.0 + p * (1.0 / 9.0)))))
    return jnp.maximum(u, 0.0) + 2.0 * ser


def _body(center_hbm, context_hbm, cids_hbm, xids_hbm, negf_hbm, out_hbm,
          cidx, xidx, nidx, crow, xrow, nrow, stage, accv, sem0, sem1):
    wid = lax.axis_index("s") * NC + lax.axis_index("c")
    base = wid * PER_W

    pltpu.sync_copy(cids_hbm.at[pl.ds(base, PER_W)], cidx)
    pltpu.sync_copy(xids_hbm.at[pl.ds(base, PER_W)], xidx)
    pltpu.sync_copy(negf_hbm.at[pl.ds(base * K, PER_W * K)], nidx)

    sems = (sem0, sem1)

    def fire(j, s):
        jc = j * C
        sem = sems[s]
        cps = [pltpu.async_copy(center_hbm.at[cidx.at[pl.ds(jc, C)]],
                                crow.at[s], sem),
               pltpu.async_copy(context_hbm.at[xidx.at[pl.ds(jc, C)]],
                                xrow.at[s], sem)]
        for off, ln in NEG_SPLITS:
            cps.append(pltpu.async_copy(
                context_hbm.at[nidx.at[pl.ds(jc * K + off, ln)]],
                nrow.at[s, pl.ds(off, ln)], sem))
        return cps

    def compute(s):
        @plsc.parallel_loop(0, C, unroll=2)
        def _(e):
            c0 = crow[s, e, pl.ds(0, 16)]
            c1 = crow[s, e, pl.ds(16, 16)]
            c2 = crow[s, e, pl.ds(32, 16)]
            c3 = crow[s, e, pl.ds(48, 16)]
            x0 = xrow[s, e, pl.ds(0, 16)]
            x1 = xrow[s, e, pl.ds(16, 16)]
            x2 = xrow[s, e, pl.ds(32, 16)]
            x3 = xrow[s, e, pl.ds(48, 16)]
            pos = c0 * x0 + c1 * x1 + c2 * x2 + c3 * x3
            base_s = e * R * 16
            stage[pl.ds(base_s, 16)] = plsc.cumsum(-pos)
            for k in range(K):
                n0 = nrow[s, e * K + k, pl.ds(0, 16)]
                n1 = nrow[s, e * K + k, pl.ds(16, 16)]
                n2 = nrow[s, e * K + k, pl.ds(32, 16)]
                n3 = nrow[s, e * K + k, pl.ds(48, 16)]
                neg = n0 * c0 + n1 * c1 + n2 * c2 + n3 * c3
                stage[pl.ds(base_s + (1 + k) * 16, 16)] = plsc.cumsum(neg)

    lane = lax.iota(jnp.int32, 16)

    def epilogue(acc):
        @plsc.parallel_loop(0, R * C // 16, unroll=2, carry=acc)
        def acc_out(g, a):
            idx = lane * 16 + (g * 256 + 15)
            v = plsc.load_gather(stage, [idx])
            u = jnp.clip(v, -10.0, 10.0)
            return a + _softplus(u)
        return acc_out

    acc = jnp.zeros((16,), jnp.float32)
    prev = fire(0, 0)
    for j in range(NCHUNK):
        nxt = fire(j + 1, (j + 1) % 2) if j + 1 < NCHUNK else []
        for cp in prev:
            cp.wait()
        compute(j % 2)
        acc = epilogue(acc)
        prev = nxt

    accv[...] = acc
    pltpu.sync_copy(accv, out_hbm.at[wid])


@jax.jit
def _sc_loss_partials(center_table, context_table, center_ids, context_ids, negf):
    mesh = plsc.VectorSubcoreMesh(core_axis_name="c", subcore_axis_name="s")
    f = pl.kernel(
        _body,
        out_type=jax.ShapeDtypeStruct((NW, 16), jnp.float32),
        mesh=mesh,
        compiler_params=pltpu.CompilerParams(
            needs_layout_passes=False, use_tc_tiling_on_sc=False),
        scratch_types=[
            pltpu.VMEM((PER_W,), jnp.int32),
            pltpu.VMEM((PER_W,), jnp.int32),
            pltpu.VMEM((K * PER_W,), jnp.int32),
            pltpu.VMEM((2, C, D), jnp.float32),
            pltpu.VMEM((2, C, D), jnp.float32),
            pltpu.VMEM((2, C * K, D), jnp.float32),
            pltpu.VMEM((R * C * 16,), jnp.float32),
            pltpu.VMEM((16,), jnp.float32),
            pltpu.SemaphoreType.DMA,
            pltpu.SemaphoreType.DMA,
        ],
    )
    return f(center_table, context_table, center_ids, context_ids, negf)


def kernel(center_table, context_table, center_ids, context_ids, neg_context_ids):
    negf = neg_context_ids.reshape(-1)
    partials = _sc_loss_partials(center_table, context_table, center_ids,
                                 context_ids, negf)
    return jnp.sum(partials) / B


# speedup vs baseline: 2.6395x; 1.5023x over previous
import jax
import jax.numpy as jnp
from jax import lax
from jax.experimental import pallas as pl
from jax.experimental.pallas import tpu as pltpu
from jax.experimental.pallas import tpu_sc as plsc

D = 64
B = 16384
K = 5
NC = 2
NS = 16
NW = NC * NS
PER_W = B // NW
C = 64
NCHUNK = PER_W // C
R = 1 + K


def _softplus(u):
    t = jnp.exp(-jnp.abs(u))
    s = t / (t + 2.0)
    p = s * s
    ser = s * (1.0 + p * (1.0 / 3.0 + p * (1.0 / 5.0 + p * (1.0 / 7.0 + p * (1.0 / 9.0)))))
    return jnp.maximum(u, 0.0) + 2.0 * ser


def _body(center_hbm, context_hbm, cids_hbm, xids_hbm, negf_hbm, out_hbm,
          cidx, xidx, nidx, crow, xrow, nrow, stage, accv, sem0, sem1):
    wid = lax.axis_index("s") * NC + lax.axis_index("c")
    base = wid * PER_W

    pltpu.sync_copy(cids_hbm.at[pl.ds(base, PER_W)], cidx)
    pltpu.sync_copy(xids_hbm.at[pl.ds(base, PER_W)], xidx)
    pltpu.sync_copy(negf_hbm.at[pl.ds(base * K, PER_W * K)], nidx)

    sems = (sem0, sem1)

    def fire(j, s):
        jc = j * C
        sem = sems[s]

        @pl.loop(0, C // 16)
        def _(g):
            civ = cidx[pl.ds(jc + g * 16, 16)]
            xiv = xidx[pl.ds(jc + g * 16, 16)]
            nvs = [nidx[pl.ds((jc + g * 16) * K + q * 16, 16)]
                   for q in range(K)]
            for i in range(16):
                e = g * 16 + i
                pltpu.async_copy(center_hbm.at[pl.ds(civ[i], 1), :],
                                 crow.at[s, pl.ds(e, 1)], sem)
                pltpu.async_copy(context_hbm.at[pl.ds(xiv[i], 1), :],
                                 xrow.at[s, pl.ds(e, 1)], sem)
                for k in range(K):
                    m = i * K + k
                    ni = nvs[m // 16][m % 16]
                    pltpu.async_copy(context_hbm.at[pl.ds(ni, 1), :],
                                     nrow.at[s, pl.ds(e * K + k, 1)], sem)

    def drain(s):
        sem = sems[s]

        @pl.loop(0, C)
        def _(e):
            pltpu.make_async_copy(center_hbm.at[pl.ds(0, 1), :],
                                  crow.at[s, pl.ds(0, 1)], sem).wait()
            pltpu.make_async_copy(context_hbm.at[pl.ds(0, 1), :],
                                  xrow.at[s, pl.ds(0, 1)], sem).wait()
            for k in range(K):
                pltpu.make_async_copy(context_hbm.at[pl.ds(0, 1), :],
                                      nrow.at[s, pl.ds(k, 1)], sem).wait()

    def compute(s):
        @plsc.parallel_loop(0, C, unroll=2)
        def _(e):
            c0 = crow[s, e, pl.ds(0, 16)]
            c1 = crow[s, e, pl.ds(16, 16)]
            c2 = crow[s, e, pl.ds(32, 16)]
            c3 = crow[s, e, pl.ds(48, 16)]
            x0 = xrow[s, e, pl.ds(0, 16)]
            x1 = xrow[s, e, pl.ds(16, 16)]
            x2 = xrow[s, e, pl.ds(32, 16)]
            x3 = xrow[s, e, pl.ds(48, 16)]
            pos = c0 * x0 + c1 * x1 + c2 * x2 + c3 * x3
            base_s = e * R * 16
            stage[pl.ds(base_s, 16)] = plsc.cumsum(-pos)
            for k in range(K):
                n0 = nrow[s, e * K + k, pl.ds(0, 16)]
                n1 = nrow[s, e * K + k, pl.ds(16, 16)]
                n2 = nrow[s, e * K + k, pl.ds(32, 16)]
                n3 = nrow[s, e * K + k, pl.ds(48, 16)]
                neg = n0 * c0 + n1 * c1 + n2 * c2 + n3 * c3
                stage[pl.ds(base_s + (1 + k) * 16, 16)] = plsc.cumsum(neg)

    lane = lax.iota(jnp.int32, 16)

    def epilogue(acc):
        @plsc.parallel_loop(0, R * C // 16, unroll=2, carry=acc)
        def acc_out(g, a):
            idx = lane * 16 + (g * 256 + 15)
            v = plsc.load_gather(stage, [idx])
            u = jnp.clip(v, -10.0, 10.0)
            return a + _softplus(u)
        return acc_out

    acc = jnp.zeros((16,), jnp.float32)
    fire(0, 0)
    for j in range(NCHUNK):
        if j + 1 < NCHUNK:
            fire(j + 1, (j + 1) % 2)
        drain(j % 2)
        compute(j % 2)
        acc = epilogue(acc)

    accv[...] = acc
    pltpu.sync_copy(accv, out_hbm.at[wid])


@jax.jit
def _sc_loss_partials(center_table, context_table, center_ids, context_ids, negf):
    mesh = plsc.VectorSubcoreMesh(core_axis_name="c", subcore_axis_name="s")
    f = pl.kernel(
        _body,
        out_type=jax.ShapeDtypeStruct((NW, 16), jnp.float32),
        mesh=mesh,
        compiler_params=pltpu.CompilerParams(
            needs_layout_passes=False, use_tc_tiling_on_sc=True),
        scratch_types=[
            pltpu.VMEM((PER_W,), jnp.int32),
            pltpu.VMEM((PER_W,), jnp.int32),
            pltpu.VMEM((K * PER_W,), jnp.int32),
            pltpu.VMEM((2, C, D), jnp.float32),
            pltpu.VMEM((2, C, D), jnp.float32),
            pltpu.VMEM((2, C * K, D), jnp.float32),
            pltpu.VMEM((R * C * 16,), jnp.float32),
            pltpu.VMEM((16,), jnp.float32),
            pltpu.SemaphoreType.DMA,
            pltpu.SemaphoreType.DMA,
        ],
    )
    return f(center_table, context_table, center_ids, context_ids, negf)


def kernel(center_table, context_table, center_ids, context_ids, neg_context_ids):
    negf = neg_context_ids.reshape(-1)
    partials = _sc_loss_partials(center_table, context_table, center_ids,
                                 context_ids, negf)
    return jnp.sum(partials) / B
